# Initial kernel scaffold; baseline (speedup 1.0000x reference)
import jax
import jax.numpy as jnp
from jax.experimental import pallas as pl


def kernel(xyz1, xyz2, fea1, fea2, W1, b1, g1, be1, W2, b2, g2, be2):
    raise NotImplementedError("write your pallas kernel here")


# speedup vs baseline: 21.7517x; 21.7517x over previous
import functools

import jax
import jax.numpy as jnp
from jax import lax
from jax.experimental import pallas as pl
from jax.experimental.pallas import tpu as pltpu
from jax.experimental.pallas import tpu_sc as plsc

TILE_A = 512
TILE_M = 512
SC_CHUNK = 64


def _knn_body(x1_ref, x2_ref, idx_ref, w_ref, *, n2):
    b = pl.program_id(0)
    x1 = x1_ref[0]
    x2 = x2_ref[0]
    dot = lax.dot_general(x1, x2, (((1,), (0,)), ((), ())),
                          preferred_element_type=jnp.float32)
    n1sq = jnp.sum(x1 * x1, axis=1, keepdims=True)
    n2sq = jnp.sum(x2 * x2, axis=0, keepdims=True)
    d = n1sq + n2sq - 2.0 * dot
    t_rows = d.shape[0]
    lanes = lax.broadcasted_iota(jnp.int32, (t_rows, n2), 1)
    big = jnp.float32(jnp.inf)

    def minarg(dd):
        m = jnp.min(dd, axis=1, keepdims=True)
        a = jnp.min(jnp.where(dd == m, lanes, n2), axis=1, keepdims=True)
        return m, a

    m1, a1 = minarg(d)
    d2 = jnp.where(lanes == a1, big, d)
    m2, a2 = minarg(d2)
    d3 = jnp.where(lanes == a2, big, d2)
    m3, a3 = minarg(d3)

    r1 = 1.0 / (m1 + 1e-8)
    r2 = 1.0 / (m2 + 1e-8)
    r3 = 1.0 / (m3 + 1e-8)
    s = r1 + r2 + r3
    w1, w2, w3 = r1 / s, r2 / s, r3 / s

    g = b * n2
    zi = jnp.zeros((t_rows, 1), jnp.int32)
    zf = jnp.zeros((t_rows, 1), jnp.float32)
    idx_ref[...] = jnp.concatenate(
        [a1 + g, a2 + g, a3 + g, zi, a1, a2, a3, zi], axis=1)
    w_ref[...] = jnp.concatenate([w1, w2, w3, zf, zf, zf, zf, zf], axis=1)


def _knn_call(xyz1, xyz2t):
    bs, n1, _ = xyz1.shape
    n2 = xyz2t.shape[2]
    nt = n1 // TILE_A
    rows = bs * n1
    return pl.pallas_call(
        functools.partial(_knn_body, n2=n2),
        grid=(bs, nt),
        in_specs=[
            pl.BlockSpec((1, TILE_A, 3), lambda b, t: (b, t, 0)),
            pl.BlockSpec((1, 3, n2), lambda b, t: (b, 0, 0)),
        ],
        out_specs=[
            pl.BlockSpec((TILE_A, 8), lambda b, t, _nt=nt: (b * _nt + t, 0)),
            pl.BlockSpec((TILE_A, 8), lambda b, t, _nt=nt: (b * _nt + t, 0)),
        ],
        out_shape=[
            jax.ShapeDtypeStruct((rows, 8), jnp.int32),
            jax.ShapeDtypeStruct((rows, 8), jnp.float32),
        ],
    )(xyz1, xyz2t)


def _sc_interp_call(fea2f, idx8, w8):
    rows, c2 = idx8.shape[0], fea2f.shape[1]
    info = plsc.get_sparse_core_info()
    nc, ns = info.num_cores, info.num_subcores
    nw = nc * ns
    per_w = rows // nw
    n_chunks = per_w // SC_CHUNK
    mesh = plsc.VectorSubcoreMesh(core_axis_name="c", subcore_axis_name="s")

    @functools.partial(
        pl.kernel, mesh=mesh,
        compiler_params=pltpu.CompilerParams(needs_layout_passes=False),
        out_type=jax.ShapeDtypeStruct((rows, c2), jnp.float32),
        scratch_types=[
            pltpu.VMEM((SC_CHUNK * 8,), jnp.int32),
            pltpu.VMEM((SC_CHUNK * 8,), jnp.float32),
            pltpu.VMEM((SC_CHUNK,), jnp.int32),
            pltpu.VMEM((SC_CHUNK,), jnp.int32),
            pltpu.VMEM((SC_CHUNK,), jnp.int32),
            pltpu.VMEM((SC_CHUNK, c2), jnp.float32),
            pltpu.VMEM((SC_CHUNK, c2), jnp.float32),
            pltpu.VMEM((SC_CHUNK, c2), jnp.float32),
            pltpu.VMEM((SC_CHUNK, c2), jnp.float32),
            pltpu.SemaphoreType.DMA,
        ],
    )
    def sc_kernel(fea2_hbm, idx_hbm, w_hbm, out_hbm,
                  idx8_v, w8_v, i1, i2, i3, r1, r2, r3, ob, sem):
        wid = lax.axis_index("s") * nc + lax.axis_index("c")
        base0 = wid * per_w

        def chunk(ci, carry):
            base = base0 + ci * SC_CHUNK
            pltpu.sync_copy(idx_hbm.at[pl.ds(base * 8, SC_CHUNK * 8)], idx8_v)
            pltpu.sync_copy(w_hbm.at[pl.ds(base * 8, SC_CHUNK * 8)], w8_v)
            for g in range(SC_CHUNK // 16):
                rr = (lax.broadcasted_iota(jnp.int32, (16,), 0) + g * 16) * 8
                sl = pl.ds(g * 16, 16)
                i1[sl] = plsc.load_gather(idx8_v, [rr])
                i2[sl] = plsc.load_gather(idx8_v, [rr + 1])
                i3[sl] = plsc.load_gather(idx8_v, [rr + 2])
            pltpu.async_copy(fea2_hbm.at[i1], r1, sem).wait()
            pltpu.async_copy(fea2_hbm.at[i2], r2, sem).wait()
            pltpu.async_copy(fea2_hbm.at[i3], r3, sem).wait()

            def pair(r, c2_):
                w1 = plsc.load_gather(w8_v, [jnp.full((16,), r * 8, jnp.int32)])
                w2 = plsc.load_gather(w8_v, [jnp.full((16,), r * 8 + 1, jnp.int32)])
                w3 = plsc.load_gather(w8_v, [jnp.full((16,), r * 8 + 2, jnp.int32)])
                for c in range(c2 // 16):
                    fsl = pl.ds(c * 16, 16)
                    acc = w1 * r1[r, fsl] + w2 * r2[r, fsl] + w3 * r3[r, fsl]
                    ob[r, fsl] = acc
                return c2_

            lax.fori_loop(0, SC_CHUNK, pair, 0)
            pltpu.sync_copy(ob, out_hbm.at[pl.ds(base, SC_CHUNK)])
            return carry

        lax.fori_loop(0, n_chunks, chunk, 0)

    return sc_kernel(fea2f, idx8.reshape(-1), w8.reshape(-1))


def _mlp1_body(f1_ref, it_ref, w1a_ref, w1b_ref, b1_ref, y_ref, st_ref):
    t = pl.program_id(0)
    y = (lax.dot_general(f1_ref[...], w1a_ref[...], (((1,), (0,)), ((), ())),
                         preferred_element_type=jnp.float32)
         + lax.dot_general(it_ref[...], w1b_ref[...], (((1,), (0,)), ((), ())),
                           preferred_element_type=jnp.float32)
         + b1_ref[...])
    y_ref[...] = y
    s = jnp.sum(y, axis=0, keepdims=True)
    q = jnp.sum(y * y, axis=0, keepdims=True)
    blk = jnp.concatenate(
        [s, q, jnp.zeros((6, y.shape[1]), jnp.float32)], axis=0)

    @pl.when(t == 0)
    def _():
        st_ref[...] = blk

    @pl.when(t > 0)
    def _():
        st_ref[...] += blk


def _mlp1_call(f1, interp, w1at, w1bt, b1r):
    rows, c1 = f1.shape
    c2 = interp.shape[1]
    co = w1at.shape[1]
    nt = rows // TILE_M
    return pl.pallas_call(
        _mlp1_body,
        grid=(nt,),
        in_specs=[
            pl.BlockSpec((TILE_M, c1), lambda t: (t, 0)),
            pl.BlockSpec((TILE_M, c2), lambda t: (t, 0)),
            pl.BlockSpec((c1, co), lambda t: (0, 0)),
            pl.BlockSpec((c2, co), lambda t: (0, 0)),
            pl.BlockSpec((1, co), lambda t: (0, 0)),
        ],
        out_specs=[
            pl.BlockSpec((TILE_M, co), lambda t: (t, 0)),
            pl.BlockSpec((8, co), lambda t: (0, 0)),
        ],
        out_shape=[
            jax.ShapeDtypeStruct((rows, co), jnp.float32),
            jax.ShapeDtypeStruct((8, co), jnp.float32),
        ],
    )(f1, interp, w1at, w1bt, b1r)


def _mlp2_body(y1_ref, st1_ref, g1_ref, be1_ref, w2t_ref, b2_ref,
               y2_ref, st_ref, *, n_rows):
    t = pl.program_id(0)
    st = st1_ref[...]
    m = st[0:1] / n_rows
    v = st[1:2] / n_rows - m * m
    a = g1_ref[...] / jnp.sqrt(v + 1e-5)
    x2 = jnp.maximum((y1_ref[...] - m) * a + be1_ref[...], 0.0)
    y2 = lax.dot_general(x2, w2t_ref[...], (((1,), (0,)), ((), ())),
                         preferred_element_type=jnp.float32) + b2_ref[...]
    y2_ref[...] = y2
    s = jnp.sum(y2, axis=0, keepdims=True)
    q = jnp.sum(y2 * y2, axis=0, keepdims=True)
    blk = jnp.concatenate(
        [s, q, jnp.zeros((6, y2.shape[1]), jnp.float32)], axis=0)

    @pl.when(t == 0)
    def _():
        st_ref[...] = blk

    @pl.when(t > 0)
    def _():
        st_ref[...] += blk


def _mlp2_call(y1, st1, g1r, be1r, w2t, b2r):
    rows, ci = y1.shape
    co = w2t.shape[1]
    nt = rows // TILE_M
    return pl.pallas_call(
        functools.partial(_mlp2_body, n_rows=float(rows)),
        grid=(nt,),
        in_specs=[
            pl.BlockSpec((TILE_M, ci), lambda t: (t, 0)),
            pl.BlockSpec((8, ci), lambda t: (0, 0)),
            pl.BlockSpec((1, ci), lambda t: (0, 0)),
            pl.BlockSpec((1, ci), lambda t: (0, 0)),
            pl.BlockSpec((ci, co), lambda t: (0, 0)),
            pl.BlockSpec((1, co), lambda t: (0, 0)),
        ],
        out_specs=[
            pl.BlockSpec((TILE_M, co), lambda t: (t, 0)),
            pl.BlockSpec((8, co), lambda t: (0, 0)),
        ],
        out_shape=[
            jax.ShapeDtypeStruct((rows, co), jnp.float32),
            jax.ShapeDtypeStruct((8, co), jnp.float32),
        ],
    )(y1, st1, g1r, be1r, w2t, b2r)


def _norm_body(y2_ref, st2_ref, g2_ref, be2_ref, o_ref, *, n_rows):
    st = st2_ref[...]
    m = st[0:1] / n_rows
    v = st[1:2] / n_rows - m * m
    a = g2_ref[...] / jnp.sqrt(v + 1e-5)
    o_ref[...] = jnp.maximum((y2_ref[...] - m) * a + be2_ref[...], 0.0)


def _norm_call(y2, st2, g2r, be2r):
    rows, co = y2.shape
    nt = rows // TILE_M
    return pl.pallas_call(
        functools.partial(_norm_body, n_rows=float(rows)),
        grid=(nt,),
        in_specs=[
            pl.BlockSpec((TILE_M, co), lambda t: (t, 0)),
            pl.BlockSpec((8, co), lambda t: (0, 0)),
            pl.BlockSpec((1, co), lambda t: (0, 0)),
            pl.BlockSpec((1, co), lambda t: (0, 0)),
        ],
        out_specs=pl.BlockSpec((TILE_M, co), lambda t: (t, 0)),
        out_shape=jax.ShapeDtypeStruct((rows, co), jnp.float32),
    )(y2, st2, g2r, be2r)


def kernel(xyz1, xyz2, fea1, fea2, W1, b1, g1, be1, W2, b2, g2, be2):
    bs, n1, _ = xyz1.shape
    n2 = xyz2.shape[1]
    c1 = fea1.shape[2]
    c2 = fea2.shape[2]
    rows = bs * n1

    xyz2t = jnp.transpose(xyz2, (0, 2, 1))
    idx8, w8 = _knn_call(xyz1, xyz2t)

    interp = _sc_interp_call(fea2.reshape(bs * n2, c2), idx8, w8)

    f1 = fea1.reshape(rows, c1)
    w1at = W1[:, :c1].T
    w1bt = W1[:, c1:].T
    y1, st1 = _mlp1_call(f1, interp, w1at, w1bt, b1.reshape(1, -1))
    y2, st2 = _mlp2_call(y1, st1, g1.reshape(1, -1), be1.reshape(1, -1),
                         W2.T, b2.reshape(1, -1))
    out = _norm_call(y2, st2, g2.reshape(1, -1), be2.reshape(1, -1))
    return out.reshape(bs, n1, W2.shape[0])


# speedup vs baseline: 22.8039x; 1.0484x over previous
import functools

import jax
import jax.numpy as jnp
from jax import lax
from jax.experimental import pallas as pl
from jax.experimental.pallas import tpu as pltpu
from jax.experimental.pallas import tpu_sc as plsc

TILE_A = 512
TILE_M = 512
SC_CHUNK = 64


def _knn_body(x1_ref, x2_ref, idx_ref, w_ref, *, n2):
    b = pl.program_id(0)
    x1 = x1_ref[0]
    x2 = x2_ref[0]
    dot = lax.dot_general(x1, x2, (((1,), (0,)), ((), ())),
                          preferred_element_type=jnp.float32)
    n1sq = jnp.sum(x1 * x1, axis=1, keepdims=True)
    n2sq = jnp.sum(x2 * x2, axis=0, keepdims=True)
    d = n1sq + n2sq - 2.0 * dot
    t_rows = d.shape[0]
    lanes = lax.broadcasted_iota(jnp.int32, (t_rows, n2), 1)
    big = jnp.float32(jnp.inf)

    def minarg(dd):
        m = jnp.min(dd, axis=1, keepdims=True)
        a = jnp.min(jnp.where(dd == m, lanes, n2), axis=1, keepdims=True)
        return m, a

    m1, a1 = minarg(d)
    d2 = jnp.where(lanes == a1, big, d)
    m2, a2 = minarg(d2)
    d3 = jnp.where(lanes == a2, big, d2)
    m3, a3 = minarg(d3)

    r1 = 1.0 / (m1 + 1e-8)
    r2 = 1.0 / (m2 + 1e-8)
    r3 = 1.0 / (m3 + 1e-8)
    s = r1 + r2 + r3
    w1, w2, w3 = r1 / s, r2 / s, r3 / s

    g = b * n2
    zi = jnp.zeros((t_rows, 1), jnp.int32)
    zf = jnp.zeros((t_rows, 1), jnp.float32)
    idx_ref[...] = jnp.concatenate(
        [a1 + g, a2 + g, a3 + g, zi, a1, a2, a3, zi], axis=1)
    w_ref[...] = jnp.concatenate([w1, w2, w3, zf, zf, zf, zf, zf], axis=1)


def _knn_call(xyz1, xyz2t):
    bs, n1, _ = xyz1.shape
    n2 = xyz2t.shape[2]
    nt = n1 // TILE_A
    rows = bs * n1
    return pl.pallas_call(
        functools.partial(_knn_body, n2=n2),
        grid=(bs, nt),
        in_specs=[
            pl.BlockSpec((1, TILE_A, 3), lambda b, t: (b, t, 0)),
            pl.BlockSpec((1, 3, n2), lambda b, t: (b, 0, 0)),
        ],
        out_specs=[
            pl.BlockSpec((TILE_A, 8), lambda b, t, _nt=nt: (b * _nt + t, 0)),
            pl.BlockSpec((TILE_A, 8), lambda b, t, _nt=nt: (b * _nt + t, 0)),
        ],
        out_shape=[
            jax.ShapeDtypeStruct((rows, 8), jnp.int32),
            jax.ShapeDtypeStruct((rows, 8), jnp.float32),
        ],
    )(xyz1, xyz2t)


def _sc_interp_call(fea2f, idx8, w8):
    rows, c2 = idx8.shape[0], fea2f.shape[1]
    info = plsc.get_sparse_core_info()
    nc, ns = info.num_cores, info.num_subcores
    nw = nc * ns
    per_w = rows // nw
    n_chunks = per_w // SC_CHUNK
    mesh = plsc.VectorSubcoreMesh(core_axis_name="c", subcore_axis_name="s")

    @functools.partial(
        pl.kernel, mesh=mesh,
        compiler_params=pltpu.CompilerParams(needs_layout_passes=False),
        out_type=jax.ShapeDtypeStruct((rows, c2), jnp.float32),
        scratch_types=[
            pltpu.VMEM((SC_CHUNK * 8,), jnp.int32),
            pltpu.VMEM((SC_CHUNK * 8,), jnp.float32),
            pltpu.VMEM((SC_CHUNK,), jnp.int32),
            pltpu.VMEM((SC_CHUNK,), jnp.int32),
            pltpu.VMEM((SC_CHUNK,), jnp.int32),
            pltpu.VMEM((SC_CHUNK, c2), jnp.float32),
            pltpu.VMEM((SC_CHUNK, c2), jnp.float32),
            pltpu.VMEM((SC_CHUNK, c2), jnp.float32),
            pltpu.VMEM((SC_CHUNK, c2), jnp.float32),
            pltpu.SemaphoreType.DMA,
        ],
    )
    def sc_kernel(fea2_hbm, idx_hbm, w_hbm, out_hbm,
                  idx8_v, w8_v, i1, i2, i3, r1, r2, r3, ob, sem):
        wid = lax.axis_index("s") * nc + lax.axis_index("c")
        base0 = wid * per_w

        def chunk(ci, carry):
            base = base0 + ci * SC_CHUNK
            pltpu.sync_copy(idx_hbm.at[pl.ds(base * 8, SC_CHUNK * 8)], idx8_v)
            pltpu.sync_copy(w_hbm.at[pl.ds(base * 8, SC_CHUNK * 8)], w8_v)
            for g in range(SC_CHUNK // 16):
                rr = (lax.broadcasted_iota(jnp.int32, (16,), 0) + g * 16) * 8
                sl = pl.ds(g * 16, 16)
                i1[sl] = plsc.load_gather(idx8_v, [rr])
                i2[sl] = plsc.load_gather(idx8_v, [rr + 1])
                i3[sl] = plsc.load_gather(idx8_v, [rr + 2])
            cp1 = pltpu.async_copy(fea2_hbm.at[i1], r1, sem)
            cp2 = pltpu.async_copy(fea2_hbm.at[i2], r2, sem)
            cp3 = pltpu.async_copy(fea2_hbm.at[i3], r3, sem)
            cp1.wait()
            cp2.wait()
            cp3.wait()

            @plsc.parallel_loop(0, SC_CHUNK, 1, unroll=2)
            def pair(r):
                w1 = plsc.load_gather(w8_v, [jnp.full((16,), r * 8, jnp.int32)])
                w2 = plsc.load_gather(w8_v, [jnp.full((16,), r * 8 + 1, jnp.int32)])
                w3 = plsc.load_gather(w8_v, [jnp.full((16,), r * 8 + 2, jnp.int32)])
                for c in range(c2 // 16):
                    fsl = pl.ds(c * 16, 16)
                    acc = w1 * r1[r, fsl] + w2 * r2[r, fsl] + w3 * r3[r, fsl]
                    ob[r, fsl] = acc
            pltpu.sync_copy(ob, out_hbm.at[pl.ds(base, SC_CHUNK)])
            return carry

        lax.fori_loop(0, n_chunks, chunk, 0)

    return sc_kernel(fea2f, idx8.reshape(-1), w8.reshape(-1))


def _mlp1_body(f1_ref, it_ref, w1a_ref, w1b_ref, b1_ref, y_ref, st_ref):
    t = pl.program_id(0)
    y = (lax.dot_general(f1_ref[...], w1a_ref[...], (((1,), (0,)), ((), ())),
                         preferred_element_type=jnp.float32)
         + lax.dot_general(it_ref[...], w1b_ref[...], (((1,), (0,)), ((), ())),
                           preferred_element_type=jnp.float32)
         + b1_ref[...])
    y_ref[...] = y
    s = jnp.sum(y, axis=0, keepdims=True)
    q = jnp.sum(y * y, axis=0, keepdims=True)
    blk = jnp.concatenate(
        [s, q, jnp.zeros((6, y.shape[1]), jnp.float32)], axis=0)

    @pl.when(t == 0)
    def _():
        st_ref[...] = blk

    @pl.when(t > 0)
    def _():
        st_ref[...] += blk


def _mlp1_call(f1, interp, w1at, w1bt, b1r):
    rows, c1 = f1.shape
    c2 = interp.shape[1]
    co = w1at.shape[1]
    nt = rows // TILE_M
    return pl.pallas_call(
        _mlp1_body,
        grid=(nt,),
        in_specs=[
            pl.BlockSpec((TILE_M, c1), lambda t: (t, 0)),
            pl.BlockSpec((TILE_M, c2), lambda t: (t, 0)),
            pl.BlockSpec((c1, co), lambda t: (0, 0)),
            pl.BlockSpec((c2, co), lambda t: (0, 0)),
            pl.BlockSpec((1, co), lambda t: (0, 0)),
        ],
        out_specs=[
            pl.BlockSpec((TILE_M, co), lambda t: (t, 0)),
            pl.BlockSpec((8, co), lambda t: (0, 0)),
        ],
        out_shape=[
            jax.ShapeDtypeStruct((rows, co), jnp.float32),
            jax.ShapeDtypeStruct((8, co), jnp.float32),
        ],
    )(f1, interp, w1at, w1bt, b1r)


def _mlp2_body(y1_ref, st1_ref, g1_ref, be1_ref, w2t_ref, b2_ref,
               y2_ref, st_ref, *, n_rows):
    t = pl.program_id(0)
    st = st1_ref[...]
    m = st[0:1] / n_rows
    v = st[1:2] / n_rows - m * m
    a = g1_ref[...] / jnp.sqrt(v + 1e-5)
    x2 = jnp.maximum((y1_ref[...] - m) * a + be1_ref[...], 0.0)
    y2 = lax.dot_general(x2, w2t_ref[...], (((1,), (0,)), ((), ())),
                         preferred_element_type=jnp.float32) + b2_ref[...]
    y2_ref[...] = y2
    s = jnp.sum(y2, axis=0, keepdims=True)
    q = jnp.sum(y2 * y2, axis=0, keepdims=True)
    blk = jnp.concatenate(
        [s, q, jnp.zeros((6, y2.shape[1]), jnp.float32)], axis=0)

    @pl.when(t == 0)
    def _():
        st_ref[...] = blk

    @pl.when(t > 0)
    def _():
        st_ref[...] += blk


def _mlp2_call(y1, st1, g1r, be1r, w2t, b2r):
    rows, ci = y1.shape
    co = w2t.shape[1]
    nt = rows // TILE_M
    return pl.pallas_call(
        functools.partial(_mlp2_body, n_rows=float(rows)),
        grid=(nt,),
        in_specs=[
            pl.BlockSpec((TILE_M, ci), lambda t: (t, 0)),
            pl.BlockSpec((8, ci), lambda t: (0, 0)),
            pl.BlockSpec((1, ci), lambda t: (0, 0)),
            pl.BlockSpec((1, ci), lambda t: (0, 0)),
            pl.BlockSpec((ci, co), lambda t: (0, 0)),
            pl.BlockSpec((1, co), lambda t: (0, 0)),
        ],
        out_specs=[
            pl.BlockSpec((TILE_M, co), lambda t: (t, 0)),
            pl.BlockSpec((8, co), lambda t: (0, 0)),
        ],
        out_shape=[
            jax.ShapeDtypeStruct((rows, co), jnp.float32),
            jax.ShapeDtypeStruct((8, co), jnp.float32),
        ],
    )(y1, st1, g1r, be1r, w2t, b2r)


def _norm_body(y2_ref, st2_ref, g2_ref, be2_ref, o_ref, *, n_rows):
    st = st2_ref[...]
    m = st[0:1] / n_rows
    v = st[1:2] / n_rows - m * m
    a = g2_ref[...] / jnp.sqrt(v + 1e-5)
    o_ref[...] = jnp.maximum((y2_ref[...] - m) * a + be2_ref[...], 0.0)


def _norm_call(y2, st2, g2r, be2r):
    rows, co = y2.shape
    nt = rows // TILE_M
    return pl.pallas_call(
        functools.partial(_norm_body, n_rows=float(rows)),
        grid=(nt,),
        in_specs=[
            pl.BlockSpec((TILE_M, co), lambda t: (t, 0)),
            pl.BlockSpec((8, co), lambda t: (0, 0)),
            pl.BlockSpec((1, co), lambda t: (0, 0)),
            pl.BlockSpec((1, co), lambda t: (0, 0)),
        ],
        out_specs=pl.BlockSpec((TILE_M, co), lambda t: (t, 0)),
        out_shape=jax.ShapeDtypeStruct((rows, co), jnp.float32),
    )(y2, st2, g2r, be2r)


def kernel(xyz1, xyz2, fea1, fea2, W1, b1, g1, be1, W2, b2, g2, be2):
    bs, n1, _ = xyz1.shape
    n2 = xyz2.shape[1]
    c1 = fea1.shape[2]
    c2 = fea2.shape[2]
    rows = bs * n1

    xyz2t = jnp.transpose(xyz2, (0, 2, 1))
    idx8, w8 = _knn_call(xyz1, xyz2t)

    interp = _sc_interp_call(fea2.reshape(bs * n2, c2), idx8, w8)

    f1 = fea1.reshape(rows, c1)
    w1at = W1[:, :c1].T
    w1bt = W1[:, c1:].T
    y1, st1 = _mlp1_call(f1, interp, w1at, w1bt, b1.reshape(1, -1))
    y2, st2 = _mlp2_call(y1, st1, g1.reshape(1, -1), be1.reshape(1, -1),
                         W2.T, b2.reshape(1, -1))
    out = _norm_call(y2, st2, g2.reshape(1, -1), be2.reshape(1, -1))
    return out.reshape(bs, n1, W2.shape[0])


# speedup vs baseline: 26.7967x; 1.1751x over previous
import functools

import jax
import jax.numpy as jnp
from jax import lax
from jax.experimental import pallas as pl
from jax.experimental.pallas import tpu as pltpu
from jax.experimental.pallas import tpu_sc as plsc

TILE_A = 512
TILE_M = 512
SC_CHUNK = 64


def _knn_body(x1_ref, x2_ref, idx_ref, w_ref, *, n2, boff):
    b = pl.program_id(0) + boff
    x1 = x1_ref[0]
    x2 = x2_ref[0]
    dot = lax.dot_general(x1, x2, (((1,), (0,)), ((), ())),
                          preferred_element_type=jnp.float32)
    n1sq = jnp.sum(x1 * x1, axis=1, keepdims=True)
    n2sq = jnp.sum(x2 * x2, axis=0, keepdims=True)
    d = n1sq + n2sq - 2.0 * dot
    t_rows = d.shape[0]
    lanes = lax.broadcasted_iota(jnp.int32, (t_rows, n2), 1)
    big = jnp.float32(jnp.inf)

    def minarg(dd):
        m = jnp.min(dd, axis=1, keepdims=True)
        a = jnp.min(jnp.where(dd == m, lanes, n2), axis=1, keepdims=True)
        return m, a

    m1, a1 = minarg(d)
    d2 = jnp.where(lanes == a1, big, d)
    m2, a2 = minarg(d2)
    d3 = jnp.where(lanes == a2, big, d2)
    m3, a3 = minarg(d3)

    r1 = 1.0 / (m1 + 1e-8)
    r2 = 1.0 / (m2 + 1e-8)
    r3 = 1.0 / (m3 + 1e-8)
    s = r1 + r2 + r3
    w1, w2, w3 = r1 / s, r2 / s, r3 / s

    g = b * n2
    zi = jnp.zeros((t_rows, 1), jnp.int32)
    zf = jnp.zeros((t_rows, 1), jnp.float32)
    idx_ref[...] = jnp.concatenate(
        [a1 + g, a2 + g, a3 + g, zi, a1, a2, a3, zi], axis=1)
    w_ref[...] = jnp.concatenate([w1, w2, w3, zf, zf, zf, zf, zf], axis=1)


def _knn_call(xyz1, xyz2t, boff):
    bs, n1, _ = xyz1.shape
    n2 = xyz2t.shape[2]
    nt = n1 // TILE_A
    rows = bs * n1
    return pl.pallas_call(
        functools.partial(_knn_body, n2=n2, boff=boff),
        grid=(bs, nt),
        in_specs=[
            pl.BlockSpec((1, TILE_A, 3), lambda b, t: (b, t, 0)),
            pl.BlockSpec((1, 3, n2), lambda b, t: (b, 0, 0)),
        ],
        out_specs=[
            pl.BlockSpec((TILE_A, 8), lambda b, t, _nt=nt: (b * _nt + t, 0)),
            pl.BlockSpec((TILE_A, 8), lambda b, t, _nt=nt: (b * _nt + t, 0)),
        ],
        out_shape=[
            jax.ShapeDtypeStruct((rows, 8), jnp.int32),
            jax.ShapeDtypeStruct((rows, 8), jnp.float32),
        ],
    )(xyz1, xyz2t)


def _sc_interp_call(fea2f, idx8, w8):
    rows, c2 = idx8.shape[0], fea2f.shape[1]
    info = plsc.get_sparse_core_info()
    nc, ns = info.num_cores, info.num_subcores
    nw = nc * ns
    per_w = rows // nw
    n_chunks = per_w // SC_CHUNK
    mesh = plsc.VectorSubcoreMesh(core_axis_name="c", subcore_axis_name="s")

    @functools.partial(
        pl.kernel, mesh=mesh,
        compiler_params=pltpu.CompilerParams(needs_layout_passes=False),
        out_type=jax.ShapeDtypeStruct((rows, c2), jnp.float32),
        scratch_types=[
            pltpu.VMEM((SC_CHUNK * 8,), jnp.int32),
            pltpu.VMEM((SC_CHUNK * 8,), jnp.float32),
            pltpu.VMEM((SC_CHUNK,), jnp.int32),
            pltpu.VMEM((SC_CHUNK,), jnp.int32),
            pltpu.VMEM((SC_CHUNK,), jnp.int32),
            pltpu.VMEM((SC_CHUNK, c2), jnp.float32),
            pltpu.VMEM((SC_CHUNK, c2), jnp.float32),
            pltpu.VMEM((SC_CHUNK, c2), jnp.float32),
            pltpu.VMEM((SC_CHUNK, c2), jnp.float32),
            pltpu.SemaphoreType.DMA,
        ],
    )
    def sc_kernel(fea2_hbm, idx_hbm, w_hbm, out_hbm,
                  idx8_v, w8_v, i1, i2, i3, r1, r2, r3, ob, sem):
        wid = lax.axis_index("s") * nc + lax.axis_index("c")
        base0 = wid * per_w

        def chunk(ci, carry):
            base = base0 + ci * SC_CHUNK
            pltpu.sync_copy(idx_hbm.at[pl.ds(base * 8, SC_CHUNK * 8)], idx8_v)
            pltpu.sync_copy(w_hbm.at[pl.ds(base * 8, SC_CHUNK * 8)], w8_v)
            for g in range(SC_CHUNK // 16):
                rr = (lax.broadcasted_iota(jnp.int32, (16,), 0) + g * 16) * 8
                sl = pl.ds(g * 16, 16)
                i1[sl] = plsc.load_gather(idx8_v, [rr])
                i2[sl] = plsc.load_gather(idx8_v, [rr + 1])
                i3[sl] = plsc.load_gather(idx8_v, [rr + 2])
            cp1 = pltpu.async_copy(fea2_hbm.at[i1], r1, sem)
            cp2 = pltpu.async_copy(fea2_hbm.at[i2], r2, sem)
            cp3 = pltpu.async_copy(fea2_hbm.at[i3], r3, sem)
            cp1.wait()
            cp2.wait()
            cp3.wait()

            @plsc.parallel_loop(0, SC_CHUNK, 1, unroll=2)
            def pair(r):
                w1 = plsc.load_gather(w8_v, [jnp.full((16,), r * 8, jnp.int32)])
                w2 = plsc.load_gather(w8_v, [jnp.full((16,), r * 8 + 1, jnp.int32)])
                w3 = plsc.load_gather(w8_v, [jnp.full((16,), r * 8 + 2, jnp.int32)])
                for c in range(c2 // 16):
                    fsl = pl.ds(c * 16, 16)
                    acc = w1 * r1[r, fsl] + w2 * r2[r, fsl] + w3 * r3[r, fsl]
                    ob[r, fsl] = acc
            pltpu.sync_copy(ob, out_hbm.at[pl.ds(base, SC_CHUNK)])
            return carry

        lax.fori_loop(0, n_chunks, chunk, 0)

    return sc_kernel(fea2f, idx8.reshape(-1), w8.reshape(-1))


def _mlp1_body(f1_ref, it0_ref, it1_ref, it2_ref, it3_ref,
               w1a_ref, w1b_ref, b1_ref, y_ref, st_ref):
    g = pl.program_id(0)
    t = pl.program_id(1)
    its = [it0_ref, it1_ref, it2_ref, it3_ref]
    it = its[0][...]
    for gi in range(1, 4):
        it = jnp.where(g == gi, its[gi][...], it)
    y = (lax.dot_general(f1_ref[...], w1a_ref[...], (((1,), (0,)), ((), ())),
                         preferred_element_type=jnp.float32)
         + lax.dot_general(it, w1b_ref[...], (((1,), (0,)), ((), ())),
                           preferred_element_type=jnp.float32)
         + b1_ref[...])
    y_ref[...] = y
    s = jnp.sum(y, axis=0, keepdims=True)
    q = jnp.sum(y * y, axis=0, keepdims=True)
    blk = jnp.concatenate(
        [s, q, jnp.zeros((6, y.shape[1]), jnp.float32)], axis=0)

    @pl.when((g == 0) & (t == 0))
    def _():
        st_ref[...] = blk

    @pl.when((g > 0) | (t > 0))
    def _():
        st_ref[...] += blk


def _mlp1_call(f1, interps, w1at, w1bt, b1r):
    rows, c1 = f1.shape
    c2 = interps[0].shape[1]
    co = w1at.shape[1]
    ng = len(interps)
    ntg = interps[0].shape[0] // TILE_M

    def piece_map(gi):
        def m(g, t):
            return (jnp.where(g == gi, t, jnp.where(g < gi, 0, ntg - 1)), 0)
        return m

    return pl.pallas_call(
        _mlp1_body,
        grid=(ng, ntg),
        in_specs=[
            pl.BlockSpec((TILE_M, c1), lambda g, t: (g * ntg + t, 0)),
            pl.BlockSpec((TILE_M, c2), piece_map(0)),
            pl.BlockSpec((TILE_M, c2), piece_map(1)),
            pl.BlockSpec((TILE_M, c2), piece_map(2)),
            pl.BlockSpec((TILE_M, c2), piece_map(3)),
            pl.BlockSpec((c1, co), lambda g, t: (0, 0)),
            pl.BlockSpec((c2, co), lambda g, t: (0, 0)),
            pl.BlockSpec((1, co), lambda g, t: (0, 0)),
        ],
        out_specs=[
            pl.BlockSpec((TILE_M, co), lambda g, t: (g * ntg + t, 0)),
            pl.BlockSpec((8, co), lambda g, t: (0, 0)),
        ],
        out_shape=[
            jax.ShapeDtypeStruct((rows, co), jnp.float32),
            jax.ShapeDtypeStruct((8, co), jnp.float32),
        ],
    )(f1, *interps, w1at, w1bt, b1r)


def _mlp2_body(y1_ref, st1_ref, g1_ref, be1_ref, w2t_ref, b2_ref,
               y2_ref, st_ref, *, n_rows):
    t = pl.program_id(0)
    st = st1_ref[...]
    m = st[0:1] / n_rows
    v = st[1:2] / n_rows - m * m
    a = g1_ref[...] / jnp.sqrt(v + 1e-5)
    x2 = jnp.maximum((y1_ref[...] - m) * a + be1_ref[...], 0.0)
    y2 = lax.dot_general(x2, w2t_ref[...], (((1,), (0,)), ((), ())),
                         preferred_element_type=jnp.float32) + b2_ref[...]
    y2_ref[...] = y2
    s = jnp.sum(y2, axis=0, keepdims=True)
    q = jnp.sum(y2 * y2, axis=0, keepdims=True)
    blk = jnp.concatenate(
        [s, q, jnp.zeros((6, y2.shape[1]), jnp.float32)], axis=0)

    @pl.when(t == 0)
    def _():
        st_ref[...] = blk

    @pl.when(t > 0)
    def _():
        st_ref[...] += blk


def _mlp2_call(y1, st1, g1r, be1r, w2t, b2r):
    rows, ci = y1.shape
    co = w2t.shape[1]
    nt = rows // TILE_M
    return pl.pallas_call(
        functools.partial(_mlp2_body, n_rows=float(rows)),
        grid=(nt,),
        in_specs=[
            pl.BlockSpec((TILE_M, ci), lambda t: (t, 0)),
            pl.BlockSpec((8, ci), lambda t: (0, 0)),
            pl.BlockSpec((1, ci), lambda t: (0, 0)),
            pl.BlockSpec((1, ci), lambda t: (0, 0)),
            pl.BlockSpec((ci, co), lambda t: (0, 0)),
            pl.BlockSpec((1, co), lambda t: (0, 0)),
        ],
        out_specs=[
            pl.BlockSpec((TILE_M, co), lambda t: (t, 0)),
            pl.BlockSpec((8, co), lambda t: (0, 0)),
        ],
        out_shape=[
            jax.ShapeDtypeStruct((rows, co), jnp.float32),
            jax.ShapeDtypeStruct((8, co), jnp.float32),
        ],
    )(y1, st1, g1r, be1r, w2t, b2r)


def _norm_body(y2_ref, st2_ref, g2_ref, be2_ref, o_ref, *, n_rows):
    st = st2_ref[...]
    m = st[0:1] / n_rows
    v = st[1:2] / n_rows - m * m
    a = g2_ref[...] / jnp.sqrt(v + 1e-5)
    o_ref[...] = jnp.maximum((y2_ref[...] - m) * a + be2_ref[...], 0.0)


def _norm_call(y2, st2, g2r, be2r):
    rows, co = y2.shape
    nt = rows // TILE_M
    return pl.pallas_call(
        functools.partial(_norm_body, n_rows=float(rows)),
        grid=(nt,),
        in_specs=[
            pl.BlockSpec((TILE_M, co), lambda t: (t, 0)),
            pl.BlockSpec((8, co), lambda t: (0, 0)),
            pl.BlockSpec((1, co), lambda t: (0, 0)),
            pl.BlockSpec((1, co), lambda t: (0, 0)),
        ],
        out_specs=pl.BlockSpec((TILE_M, co), lambda t: (t, 0)),
        out_shape=jax.ShapeDtypeStruct((rows, co), jnp.float32),
    )(y2, st2, g2r, be2r)


def kernel(xyz1, xyz2, fea1, fea2, W1, b1, g1, be1, W2, b2, g2, be2):
    bs, n1, _ = xyz1.shape
    n2 = xyz2.shape[1]
    c1 = fea1.shape[2]
    c2 = fea2.shape[2]
    rows = bs * n1

    xyz2t = jnp.transpose(xyz2, (0, 2, 1))
    fea2f = fea2.reshape(bs * n2, c2)

    ng = 4
    bg = bs // ng
    interps = []
    for g in range(ng):
        sl = slice(g * bg, (g + 1) * bg)
        idx8, w8 = _knn_call(xyz1[sl], xyz2t[sl], g * bg)
        interps.append(_sc_interp_call(fea2f, idx8, w8))

    f1 = fea1.reshape(rows, c1)
    w1at = W1[:, :c1].T
    w1bt = W1[:, c1:].T
    y1, st1 = _mlp1_call(f1, interps, w1at, w1bt, b1.reshape(1, -1))
    y2, st2 = _mlp2_call(y1, st1, g1.reshape(1, -1), be1.reshape(1, -1),
                         W2.T, b2.reshape(1, -1))
    out = _norm_call(y2, st2, g2.reshape(1, -1), be2.reshape(1, -1))
    return out.reshape(bs, n1, W2.shape[0])


# speedup vs baseline: 27.8180x; 1.0381x over previous
import functools

import jax
import jax.numpy as jnp
from jax import lax
from jax.experimental import pallas as pl
from jax.experimental.pallas import tpu as pltpu
from jax.experimental.pallas import tpu_sc as plsc

TILE_A = 512
TILE_M = 512
SC_CHUNK = 64


def _knn_body(x1_ref, x2_ref, idx_ref, w_ref, *, n2, boff):
    b = pl.program_id(0) + boff
    x1 = x1_ref[0]
    x2 = x2_ref[0]
    dot = lax.dot_general(x1, x2, (((1,), (0,)), ((), ())),
                          preferred_element_type=jnp.float32)
    n1sq = jnp.sum(x1 * x1, axis=1, keepdims=True)
    n2sq = jnp.sum(x2 * x2, axis=0, keepdims=True)
    d = n1sq + n2sq - 2.0 * dot
    t_rows = d.shape[0]
    lanes = lax.broadcasted_iota(jnp.int32, (t_rows, n2), 1)
    big = jnp.float32(jnp.inf)

    def minarg(dd):
        m = jnp.min(dd, axis=1, keepdims=True)
        a = jnp.min(jnp.where(dd == m, lanes, n2), axis=1, keepdims=True)
        return m, a

    m1, a1 = minarg(d)
    d2 = jnp.where(lanes == a1, big, d)
    m2, a2 = minarg(d2)
    d3 = jnp.where(lanes == a2, big, d2)
    m3, a3 = minarg(d3)

    r1 = 1.0 / (m1 + 1e-8)
    r2 = 1.0 / (m2 + 1e-8)
    r3 = 1.0 / (m3 + 1e-8)
    s = r1 + r2 + r3
    w1, w2, w3 = r1 / s, r2 / s, r3 / s

    g = b * n2
    zi = jnp.zeros((t_rows, 1), jnp.int32)
    zf = jnp.zeros((t_rows, 1), jnp.float32)
    idx_ref[...] = jnp.concatenate(
        [a1 + g, a2 + g, a3 + g, zi, a1, a2, a3, zi], axis=1)
    w_ref[...] = jnp.concatenate([w1, w2, w3, zf, zf, zf, zf, zf], axis=1)


def _knn_call(xyz1, xyz2t, boff):
    bs, n1, _ = xyz1.shape
    n2 = xyz2t.shape[2]
    nt = n1 // TILE_A
    rows = bs * n1
    return pl.pallas_call(
        functools.partial(_knn_body, n2=n2, boff=boff),
        grid=(bs, nt),
        in_specs=[
            pl.BlockSpec((1, TILE_A, 3), lambda b, t: (b, t, 0)),
            pl.BlockSpec((1, 3, n2), lambda b, t: (b, 0, 0)),
        ],
        out_specs=[
            pl.BlockSpec((TILE_A, 8), lambda b, t, _nt=nt: (b * _nt + t, 0)),
            pl.BlockSpec((TILE_A, 8), lambda b, t, _nt=nt: (b * _nt + t, 0)),
        ],
        out_shape=[
            jax.ShapeDtypeStruct((rows, 8), jnp.int32),
            jax.ShapeDtypeStruct((rows, 8), jnp.float32),
        ],
    )(xyz1, xyz2t)


def _sc_interp_call(fea2f, idx8, w8):
    rows, c2 = idx8.shape[0], fea2f.shape[1]
    info = plsc.get_sparse_core_info()
    nc, ns = info.num_cores, info.num_subcores
    nw = nc * ns
    per_w = rows // nw
    n_chunks = per_w // SC_CHUNK
    mesh = plsc.VectorSubcoreMesh(core_axis_name="c", subcore_axis_name="s")

    @functools.partial(
        pl.kernel, mesh=mesh,
        compiler_params=pltpu.CompilerParams(needs_layout_passes=False),
        out_type=jax.ShapeDtypeStruct((rows, c2), jnp.float32),
        scratch_types=[
            pltpu.VMEM((SC_CHUNK * 8,), jnp.int32),
            pltpu.VMEM((SC_CHUNK * 8,), jnp.float32),
            pltpu.VMEM((SC_CHUNK,), jnp.int32),
            pltpu.VMEM((SC_CHUNK,), jnp.int32),
            pltpu.VMEM((SC_CHUNK,), jnp.int32),
            pltpu.VMEM((SC_CHUNK, c2), jnp.float32),
            pltpu.VMEM((SC_CHUNK, c2), jnp.float32),
            pltpu.VMEM((SC_CHUNK, c2), jnp.float32),
            pltpu.VMEM((SC_CHUNK, c2), jnp.float32),
            pltpu.SemaphoreType.DMA,
        ],
    )
    def sc_kernel(fea2_hbm, idx_hbm, w_hbm, out_hbm,
                  idx8_v, w8_v, i1, i2, i3, r1, r2, r3, ob, sem):
        wid = lax.axis_index("s") * nc + lax.axis_index("c")
        base0 = wid * per_w

        def chunk(ci, carry):
            base = base0 + ci * SC_CHUNK
            pltpu.sync_copy(idx_hbm.at[pl.ds(base * 8, SC_CHUNK * 8)], idx8_v)
            pltpu.sync_copy(w_hbm.at[pl.ds(base * 8, SC_CHUNK * 8)], w8_v)
            for g in range(SC_CHUNK // 16):
                rr = (lax.broadcasted_iota(jnp.int32, (16,), 0) + g * 16) * 8
                sl = pl.ds(g * 16, 16)
                i1[sl] = plsc.load_gather(idx8_v, [rr])
                i2[sl] = plsc.load_gather(idx8_v, [rr + 1])
                i3[sl] = plsc.load_gather(idx8_v, [rr + 2])
            cp1 = pltpu.async_copy(fea2_hbm.at[i1], r1, sem)
            cp2 = pltpu.async_copy(fea2_hbm.at[i2], r2, sem)
            cp3 = pltpu.async_copy(fea2_hbm.at[i3], r3, sem)
            cp1.wait()
            cp2.wait()
            cp3.wait()

            @plsc.parallel_loop(0, SC_CHUNK, 1, unroll=2)
            def pair(r):
                w1 = plsc.load_gather(w8_v, [jnp.full((16,), r * 8, jnp.int32)])
                w2 = plsc.load_gather(w8_v, [jnp.full((16,), r * 8 + 1, jnp.int32)])
                w3 = plsc.load_gather(w8_v, [jnp.full((16,), r * 8 + 2, jnp.int32)])
                for c in range(c2 // 16):
                    fsl = pl.ds(c * 16, 16)
                    acc = w1 * r1[r, fsl] + w2 * r2[r, fsl] + w3 * r3[r, fsl]
                    ob[r, fsl] = acc
            pltpu.sync_copy(ob, out_hbm.at[pl.ds(base, SC_CHUNK)])
            return carry

        lax.fori_loop(0, n_chunks, chunk, 0)

    return sc_kernel(fea2f, idx8.reshape(-1), w8.reshape(-1))


def _mlp1_body(f1_ref, it0_ref, it1_ref, it2_ref, it3_ref,
               w1a_ref, w1b_ref, b1_ref, y_ref, st_ref):
    g = pl.program_id(0)
    t = pl.program_id(1)
    its = [it0_ref, it1_ref, it2_ref, it3_ref]
    it = its[0][...]
    for gi in range(1, 4):
        it = jnp.where(g == gi, its[gi][...], it)
    y = (lax.dot_general(f1_ref[...], w1a_ref[...], (((1,), (0,)), ((), ())),
                         preferred_element_type=jnp.float32)
         + lax.dot_general(it, w1b_ref[...], (((1,), (0,)), ((), ())),
                           preferred_element_type=jnp.float32)
         + b1_ref[...])
    y_ref[...] = y.astype(jnp.bfloat16)
    s = jnp.sum(y, axis=0, keepdims=True)
    q = jnp.sum(y * y, axis=0, keepdims=True)
    blk = jnp.concatenate(
        [s, q, jnp.zeros((6, y.shape[1]), jnp.float32)], axis=0)

    @pl.when((g == 0) & (t == 0))
    def _():
        st_ref[...] = blk

    @pl.when((g > 0) | (t > 0))
    def _():
        st_ref[...] += blk


def _mlp1_call(f1, interps, w1at, w1bt, b1r):
    rows, c1 = f1.shape
    c2 = interps[0].shape[1]
    co = w1at.shape[1]
    ng = len(interps)
    ntg = interps[0].shape[0] // TILE_M

    def piece_map(gi):
        def m(g, t):
            return (jnp.where(g == gi, t, jnp.where(g < gi, 0, ntg - 1)), 0)
        return m

    return pl.pallas_call(
        _mlp1_body,
        grid=(ng, ntg),
        in_specs=[
            pl.BlockSpec((TILE_M, c1), lambda g, t: (g * ntg + t, 0)),
            pl.BlockSpec((TILE_M, c2), piece_map(0)),
            pl.BlockSpec((TILE_M, c2), piece_map(1)),
            pl.BlockSpec((TILE_M, c2), piece_map(2)),
            pl.BlockSpec((TILE_M, c2), piece_map(3)),
            pl.BlockSpec((c1, co), lambda g, t: (0, 0)),
            pl.BlockSpec((c2, co), lambda g, t: (0, 0)),
            pl.BlockSpec((1, co), lambda g, t: (0, 0)),
        ],
        out_specs=[
            pl.BlockSpec((TILE_M, co), lambda g, t: (g * ntg + t, 0)),
            pl.BlockSpec((8, co), lambda g, t: (0, 0)),
        ],
        out_shape=[
            jax.ShapeDtypeStruct((rows, co), jnp.bfloat16),
            jax.ShapeDtypeStruct((8, co), jnp.float32),
        ],
    )(f1, *interps, w1at, w1bt, b1r)


def _mlp2_body(y1_ref, st1_ref, g1_ref, be1_ref, w2t_ref, b2_ref,
               y2_ref, st_ref, *, n_rows):
    t = pl.program_id(0)
    st = st1_ref[...]
    m = st[0:1] / n_rows
    v = st[1:2] / n_rows - m * m
    a = g1_ref[...] / jnp.sqrt(v + 1e-5)
    y1 = y1_ref[...].astype(jnp.float32)
    x2 = jnp.maximum((y1 - m) * a + be1_ref[...], 0.0)
    y2 = lax.dot_general(x2, w2t_ref[...], (((1,), (0,)), ((), ())),
                         preferred_element_type=jnp.float32) + b2_ref[...]
    y2_ref[...] = y2.astype(jnp.bfloat16)
    s = jnp.sum(y2, axis=0, keepdims=True)
    q = jnp.sum(y2 * y2, axis=0, keepdims=True)
    blk = jnp.concatenate(
        [s, q, jnp.zeros((6, y2.shape[1]), jnp.float32)], axis=0)

    @pl.when(t == 0)
    def _():
        st_ref[...] = blk

    @pl.when(t > 0)
    def _():
        st_ref[...] += blk


def _mlp2_call(y1, st1, g1r, be1r, w2t, b2r):
    rows, ci = y1.shape
    co = w2t.shape[1]
    nt = rows // TILE_M
    return pl.pallas_call(
        functools.partial(_mlp2_body, n_rows=float(rows)),
        grid=(nt,),
        in_specs=[
            pl.BlockSpec((TILE_M, ci), lambda t: (t, 0)),
            pl.BlockSpec((8, ci), lambda t: (0, 0)),
            pl.BlockSpec((1, ci), lambda t: (0, 0)),
            pl.BlockSpec((1, ci), lambda t: (0, 0)),
            pl.BlockSpec((ci, co), lambda t: (0, 0)),
            pl.BlockSpec((1, co), lambda t: (0, 0)),
        ],
        out_specs=[
            pl.BlockSpec((TILE_M, co), lambda t: (t, 0)),
            pl.BlockSpec((8, co), lambda t: (0, 0)),
        ],
        out_shape=[
            jax.ShapeDtypeStruct((rows, co), jnp.bfloat16),
            jax.ShapeDtypeStruct((8, co), jnp.float32),
        ],
    )(y1, st1, g1r, be1r, w2t, b2r)


def _norm_body(y2_ref, st2_ref, g2_ref, be2_ref, o_ref, *, n_rows):
    st = st2_ref[...]
    m = st[0:1] / n_rows
    v = st[1:2] / n_rows - m * m
    a = g2_ref[...] / jnp.sqrt(v + 1e-5)
    y2 = y2_ref[...].astype(jnp.float32)
    o_ref[...] = jnp.maximum((y2 - m) * a + be2_ref[...], 0.0)


def _norm_call(y2, st2, g2r, be2r):
    rows, co = y2.shape
    nt = rows // TILE_M
    return pl.pallas_call(
        functools.partial(_norm_body, n_rows=float(rows)),
        grid=(nt,),
        in_specs=[
            pl.BlockSpec((TILE_M, co), lambda t: (t, 0)),
            pl.BlockSpec((8, co), lambda t: (0, 0)),
            pl.BlockSpec((1, co), lambda t: (0, 0)),
            pl.BlockSpec((1, co), lambda t: (0, 0)),
        ],
        out_specs=pl.BlockSpec((TILE_M, co), lambda t: (t, 0)),
        out_shape=jax.ShapeDtypeStruct((rows, co), jnp.float32),
    )(y2, st2, g2r, be2r)


def kernel(xyz1, xyz2, fea1, fea2, W1, b1, g1, be1, W2, b2, g2, be2):
    bs, n1, _ = xyz1.shape
    n2 = xyz2.shape[1]
    c1 = fea1.shape[2]
    c2 = fea2.shape[2]
    rows = bs * n1

    xyz2t = jnp.transpose(xyz2, (0, 2, 1))
    fea2f = fea2.reshape(bs * n2, c2)

    ng = 4
    bg = bs // ng
    interps = []
    for g in range(ng):
        sl = slice(g * bg, (g + 1) * bg)
        idx8, w8 = _knn_call(xyz1[sl], xyz2t[sl], g * bg)
        interps.append(_sc_interp_call(fea2f, idx8, w8))

    f1 = fea1.reshape(rows, c1)
    w1at = W1[:, :c1].T
    w1bt = W1[:, c1:].T
    y1, st1 = _mlp1_call(f1, interps, w1at, w1bt, b1.reshape(1, -1))
    y2, st2 = _mlp2_call(y1, st1, g1.reshape(1, -1), be1.reshape(1, -1),
                         W2.T, b2.reshape(1, -1))
    out = _norm_call(y2, st2, g2.reshape(1, -1), be2.reshape(1, -1))
    return out.reshape(bs, n1, W2.shape[0])


# speedup vs baseline: 30.7094x; 1.1039x over previous
import functools

import jax
import jax.numpy as jnp
from jax import lax
from jax.experimental import pallas as pl
from jax.experimental.pallas import tpu as pltpu
from jax.experimental.pallas import tpu_sc as plsc

TILE_A = 512
TILE_M = 512
SC_CHUNK = 64


def _knn_body(x1_ref, x2_ref, idx_ref, w_ref, *, n2, boff):
    b = pl.program_id(0) + boff
    x1 = x1_ref[0]
    x2 = x2_ref[0]
    dot = lax.dot_general(x2, x1, (((1,), (0,)), ((), ())),
                          preferred_element_type=jnp.float32)
    n1sq = jnp.sum(x1 * x1, axis=0, keepdims=True)
    n2sq = jnp.sum(x2 * x2, axis=1, keepdims=True)
    d = n1sq + n2sq - 2.0 * dot
    t_cols = d.shape[1]
    cand = lax.broadcasted_iota(jnp.int32, (n2, t_cols), 0).astype(jnp.float32)
    big = jnp.float32(jnp.inf)
    bigc = jnp.float32(n2)

    def minarg(dd):
        m = jnp.min(dd, axis=0, keepdims=True)
        a = jnp.min(jnp.where(dd == m, cand, bigc), axis=0, keepdims=True)
        return m, a

    m1, a1f = minarg(d)
    d2 = jnp.where(cand == a1f, big, d)
    m2, a2f = minarg(d2)
    d3 = jnp.where(cand == a2f, big, d2)
    m3, a3f = minarg(d3)
    a1 = a1f.astype(jnp.int32)
    a2 = a2f.astype(jnp.int32)
    a3 = a3f.astype(jnp.int32)

    r1 = 1.0 / (m1 + 1e-8)
    r2 = 1.0 / (m2 + 1e-8)
    r3 = 1.0 / (m3 + 1e-8)
    s = r1 + r2 + r3
    w1, w2, w3 = r1 / s, r2 / s, r3 / s

    g = b * n2
    zi = jnp.zeros((5, t_cols), jnp.int32)
    zf = jnp.zeros((5, t_cols), jnp.float32)
    idx_ref[0] = jnp.concatenate([a1 + g, a2 + g, a3 + g, zi], axis=0)
    w_ref[0] = jnp.concatenate([w1, w2, w3, zf], axis=0)


def _knn_call(xyz1p, xyz2p, boff):
    bs, n1 = xyz1p.shape[0], xyz1p.shape[2]
    n2 = xyz2p.shape[1]
    nt = n1 // TILE_A
    rows = bs * n1
    return pl.pallas_call(
        functools.partial(_knn_body, n2=n2, boff=boff),
        grid=(bs, nt),
        in_specs=[
            pl.BlockSpec((1, 8, TILE_A), lambda b, t: (b, 0, t)),
            pl.BlockSpec((1, n2, 8), lambda b, t: (b, 0, 0)),
        ],
        out_specs=[
            pl.BlockSpec((1, 8, TILE_A), lambda b, t, _nt=nt: (0, 0, b * _nt + t)),
            pl.BlockSpec((1, 8, TILE_A), lambda b, t, _nt=nt: (0, 0, b * _nt + t)),
        ],
        out_shape=[
            jax.ShapeDtypeStruct((1, 8, rows), jnp.int32),
            jax.ShapeDtypeStruct((1, 8, rows), jnp.float32),
        ],
    )(xyz1p, xyz2p)


def _sc_interp_call(fea2f, idx8, w8):
    rows, c2 = idx8.shape[1], fea2f.shape[1]
    info = plsc.get_sparse_core_info()
    nc, ns = info.num_cores, info.num_subcores
    nw = nc * ns
    per_w = rows // nw
    n_chunks = per_w // SC_CHUNK
    mesh = plsc.VectorSubcoreMesh(core_axis_name="c", subcore_axis_name="s")

    @functools.partial(
        pl.kernel, mesh=mesh,
        compiler_params=pltpu.CompilerParams(needs_layout_passes=False),
        out_type=jax.ShapeDtypeStruct((rows, c2), jnp.float32),
        scratch_types=[
            pltpu.VMEM((SC_CHUNK,), jnp.int32),
            pltpu.VMEM((SC_CHUNK,), jnp.int32),
            pltpu.VMEM((SC_CHUNK,), jnp.int32),
            pltpu.VMEM((SC_CHUNK,), jnp.float32),
            pltpu.VMEM((SC_CHUNK,), jnp.float32),
            pltpu.VMEM((SC_CHUNK,), jnp.float32),
            pltpu.VMEM((SC_CHUNK, c2), jnp.float32),
            pltpu.VMEM((SC_CHUNK, c2), jnp.float32),
            pltpu.VMEM((SC_CHUNK, c2), jnp.float32),
            pltpu.VMEM((SC_CHUNK, c2), jnp.float32),
            pltpu.SemaphoreType.DMA,
        ],
    )
    def sc_kernel(fea2_hbm, idx_hbm, w_hbm, out_hbm,
                  i1, i2, i3, wv1, wv2, wv3, r1, r2, r3, ob, sem):
        wid = lax.axis_index("s") * nc + lax.axis_index("c")
        base0 = wid * per_w

        def chunk(ci, carry):
            base = base0 + ci * SC_CHUNK
            sl = pl.ds(base, SC_CHUNK)
            pltpu.sync_copy(idx_hbm.at[0, sl], i1)
            pltpu.sync_copy(idx_hbm.at[1, sl], i2)
            pltpu.sync_copy(idx_hbm.at[2, sl], i3)
            pltpu.sync_copy(w_hbm.at[0, sl], wv1)
            pltpu.sync_copy(w_hbm.at[1, sl], wv2)
            pltpu.sync_copy(w_hbm.at[2, sl], wv3)
            cp1 = pltpu.async_copy(fea2_hbm.at[i1], r1, sem)
            cp2 = pltpu.async_copy(fea2_hbm.at[i2], r2, sem)
            cp3 = pltpu.async_copy(fea2_hbm.at[i3], r3, sem)
            cp1.wait()
            cp2.wait()
            cp3.wait()

            @plsc.parallel_loop(0, SC_CHUNK, 1, unroll=2)
            def pair(r):
                w1 = plsc.load_gather(wv1, [jnp.full((16,), r, jnp.int32)])
                w2 = plsc.load_gather(wv2, [jnp.full((16,), r, jnp.int32)])
                w3 = plsc.load_gather(wv3, [jnp.full((16,), r, jnp.int32)])
                for c in range(c2 // 16):
                    fsl = pl.ds(c * 16, 16)
                    acc = w1 * r1[r, fsl] + w2 * r2[r, fsl] + w3 * r3[r, fsl]
                    ob[r, fsl] = acc
            pltpu.sync_copy(ob, out_hbm.at[pl.ds(base, SC_CHUNK)])
            return carry

        lax.fori_loop(0, n_chunks, chunk, 0)

    return sc_kernel(fea2f, idx8, w8)


def _mlp1_body(f1_ref, it0_ref, it1_ref, it2_ref, it3_ref,
               w1a_ref, w1b_ref, b1_ref, y_ref, st_ref, it_scr):
    g = pl.program_id(0)
    t = pl.program_id(1)
    its = [it0_ref, it1_ref, it2_ref, it3_ref]
    for gi in range(4):
        @pl.when(g == gi)
        def _(gi=gi):
            it_scr[...] = its[gi][...]
    it = it_scr[...]
    y = (lax.dot_general(f1_ref[...], w1a_ref[...], (((1,), (0,)), ((), ())),
                         preferred_element_type=jnp.float32)
         + lax.dot_general(it, w1b_ref[...], (((1,), (0,)), ((), ())),
                           preferred_element_type=jnp.float32)
         + b1_ref[...])
    y_ref[...] = y.astype(jnp.bfloat16)
    s = jnp.sum(y, axis=0, keepdims=True)
    q = jnp.sum(y * y, axis=0, keepdims=True)
    blk = jnp.concatenate(
        [s, q, jnp.zeros((6, y.shape[1]), jnp.float32)], axis=0)

    @pl.when((g == 0) & (t == 0))
    def _():
        st_ref[...] = blk

    @pl.when((g > 0) | (t > 0))
    def _():
        st_ref[...] += blk


def _mlp1_call(f1, interps, w1at, w1bt, b1r):
    rows, c1 = f1.shape
    c2 = interps[0].shape[1]
    co = w1at.shape[1]
    ng = len(interps)
    ntg = interps[0].shape[0] // TILE_M

    def piece_map(gi):
        def m(g, t):
            return (jnp.where(g == gi, t, jnp.where(g < gi, 0, ntg - 1)), 0)
        return m

    return pl.pallas_call(
        _mlp1_body,
        grid=(ng, ntg),
        in_specs=[
            pl.BlockSpec((TILE_M, c1), lambda g, t: (g * ntg + t, 0)),
            pl.BlockSpec((TILE_M, c2), piece_map(0)),
            pl.BlockSpec((TILE_M, c2), piece_map(1)),
            pl.BlockSpec((TILE_M, c2), piece_map(2)),
            pl.BlockSpec((TILE_M, c2), piece_map(3)),
            pl.BlockSpec((c1, co), lambda g, t: (0, 0)),
            pl.BlockSpec((c2, co), lambda g, t: (0, 0)),
            pl.BlockSpec((1, co), lambda g, t: (0, 0)),
        ],
        out_specs=[
            pl.BlockSpec((TILE_M, co), lambda g, t: (g * ntg + t, 0)),
            pl.BlockSpec((8, co), lambda g, t: (0, 0)),
        ],
        out_shape=[
            jax.ShapeDtypeStruct((rows, co), jnp.bfloat16),
            jax.ShapeDtypeStruct((8, co), jnp.float32),
        ],
        scratch_shapes=[pltpu.VMEM((TILE_M, c2), jnp.float32)],
    )(f1, *interps, w1at, w1bt, b1r)


def _mlp2_body(y1_ref, st1_ref, g1_ref, be1_ref, w2t_ref, b2_ref,
               y2_ref, st_ref, *, n_rows):
    t = pl.program_id(0)
    st = st1_ref[...]
    m = st[0:1] / n_rows
    v = st[1:2] / n_rows - m * m
    a = g1_ref[...] / jnp.sqrt(v + 1e-5)
    y1 = y1_ref[...].astype(jnp.float32)
    x2 = jnp.maximum((y1 - m) * a + be1_ref[...], 0.0)
    y2 = lax.dot_general(x2, w2t_ref[...], (((1,), (0,)), ((), ())),
                         preferred_element_type=jnp.float32) + b2_ref[...]
    y2_ref[...] = y2.astype(jnp.bfloat16)
    s = jnp.sum(y2, axis=0, keepdims=True)
    q = jnp.sum(y2 * y2, axis=0, keepdims=True)
    blk = jnp.concatenate(
        [s, q, jnp.zeros((6, y2.shape[1]), jnp.float32)], axis=0)

    @pl.when(t == 0)
    def _():
        st_ref[...] = blk

    @pl.when(t > 0)
    def _():
        st_ref[...] += blk


def _mlp2_call(y1, st1, g1r, be1r, w2t, b2r):
    rows, ci = y1.shape
    co = w2t.shape[1]
    nt = rows // TILE_M
    return pl.pallas_call(
        functools.partial(_mlp2_body, n_rows=float(rows)),
        grid=(nt,),
        in_specs=[
            pl.BlockSpec((TILE_M, ci), lambda t: (t, 0)),
            pl.BlockSpec((8, ci), lambda t: (0, 0)),
            pl.BlockSpec((1, ci), lambda t: (0, 0)),
            pl.BlockSpec((1, ci), lambda t: (0, 0)),
            pl.BlockSpec((ci, co), lambda t: (0, 0)),
            pl.BlockSpec((1, co), lambda t: (0, 0)),
        ],
        out_specs=[
            pl.BlockSpec((TILE_M, co), lambda t: (t, 0)),
            pl.BlockSpec((8, co), lambda t: (0, 0)),
        ],
        out_shape=[
            jax.ShapeDtypeStruct((rows, co), jnp.bfloat16),
            jax.ShapeDtypeStruct((8, co), jnp.float32),
        ],
    )(y1, st1, g1r, be1r, w2t, b2r)


def _norm_body(y2_ref, st2_ref, g2_ref, be2_ref, o_ref, *, n_rows):
    st = st2_ref[...]
    m = st[0:1] / n_rows
    v = st[1:2] / n_rows - m * m
    a = g2_ref[...] / jnp.sqrt(v + 1e-5)
    y2 = y2_ref[...].astype(jnp.float32)
    o_ref[...] = jnp.maximum((y2 - m) * a + be2_ref[...], 0.0)


def _norm_call(y2, st2, g2r, be2r):
    rows, co = y2.shape
    nt = rows // TILE_M
    return pl.pallas_call(
        functools.partial(_norm_body, n_rows=float(rows)),
        grid=(nt,),
        in_specs=[
            pl.BlockSpec((TILE_M, co), lambda t: (t, 0)),
            pl.BlockSpec((8, co), lambda t: (0, 0)),
            pl.BlockSpec((1, co), lambda t: (0, 0)),
            pl.BlockSpec((1, co), lambda t: (0, 0)),
        ],
        out_specs=pl.BlockSpec((TILE_M, co), lambda t: (t, 0)),
        out_shape=jax.ShapeDtypeStruct((rows, co), jnp.float32),
    )(y2, st2, g2r, be2r)


def kernel(xyz1, xyz2, fea1, fea2, W1, b1, g1, be1, W2, b2, g2, be2):
    bs, n1, _ = xyz1.shape
    n2 = xyz2.shape[1]
    c1 = fea1.shape[2]
    c2 = fea2.shape[2]
    rows = bs * n1

    xyz1p = jnp.concatenate(
        [jnp.transpose(xyz1, (0, 2, 1)),
         jnp.zeros((bs, 5, n1), jnp.float32)], axis=1)
    xyz2p = jnp.concatenate(
        [xyz2, jnp.zeros((bs, n2, 5), jnp.float32)], axis=2)
    fea2f = fea2.reshape(bs * n2, c2)

    ng = 4
    bg = bs // ng
    interps = []
    for g in range(ng):
        sl = slice(g * bg, (g + 1) * bg)
        idx8, w8 = _knn_call(xyz1p[sl], xyz2p[sl], g * bg)
        interps.append(_sc_interp_call(fea2f, idx8[0], w8[0]))

    f1 = fea1.reshape(rows, c1)
    w1at = W1[:, :c1].T
    w1bt = W1[:, c1:].T
    y1, st1 = _mlp1_call(f1, interps, w1at, w1bt, b1.reshape(1, -1))
    y2, st2 = _mlp2_call(y1, st1, g1.reshape(1, -1), be1.reshape(1, -1),
                         W2.T, b2.reshape(1, -1))
    out = _norm_call(y2, st2, g2.reshape(1, -1), be2.reshape(1, -1))
    return out.reshape(bs, n1, W2.shape[0])


# speedup vs baseline: 36.7778x; 1.1976x over previous
import functools

import jax
import jax.numpy as jnp
from jax import lax
from jax.experimental import pallas as pl
from jax.experimental.pallas import tpu as pltpu
from jax.experimental.pallas import tpu_sc as plsc

TILE_A = 1024
TILE_M = 1024
SC_CHUNK = 64


def _knn_body(x1_ref, x2_ref, idx_ref, w_ref, *, n2, boff):
    b = pl.program_id(0) + boff
    x1 = x1_ref[0]
    x2 = x2_ref[0]
    dot = lax.dot_general(x2, x1, (((1,), (0,)), ((), ())),
                          preferred_element_type=jnp.float32)
    n1sq = jnp.sum(x1 * x1, axis=0, keepdims=True)
    n2sq = jnp.sum(x2 * x2, axis=1, keepdims=True)
    d = n1sq + n2sq - 2.0 * dot
    t_cols = d.shape[1]
    cand = lax.broadcasted_iota(jnp.int32, (n2, t_cols), 0).astype(jnp.float32)
    big = jnp.float32(jnp.inf)
    bigc = jnp.float32(n2)

    def minarg(dd):
        m = jnp.min(dd, axis=0, keepdims=True)
        a = jnp.min(jnp.where(dd == m, cand, bigc), axis=0, keepdims=True)
        return m, a

    m1, a1f = minarg(d)
    d2 = jnp.where(cand == a1f, big, d)
    m2, a2f = minarg(d2)
    d3 = jnp.where(cand == a2f, big, d2)
    m3, a3f = minarg(d3)
    a1 = a1f.astype(jnp.int32)
    a2 = a2f.astype(jnp.int32)
    a3 = a3f.astype(jnp.int32)

    r1 = 1.0 / (m1 + 1e-8)
    r2 = 1.0 / (m2 + 1e-8)
    r3 = 1.0 / (m3 + 1e-8)
    s = r1 + r2 + r3
    w1, w2, w3 = r1 / s, r2 / s, r3 / s

    g = b * n2
    zi = jnp.zeros((5, t_cols), jnp.int32)
    zf = jnp.zeros((5, t_cols), jnp.float32)
    idx_ref[0] = jnp.concatenate([a1 + g, a2 + g, a3 + g, zi], axis=0)
    w_ref[0] = jnp.concatenate([w1, w2, w3, zf], axis=0)


def _knn_call(xyz1p, xyz2p, boff):
    bs, n1 = xyz1p.shape[0], xyz1p.shape[2]
    n2 = xyz2p.shape[1]
    ta = min(TILE_A, n1)
    nt = n1 // ta
    rows = bs * n1
    return pl.pallas_call(
        functools.partial(_knn_body, n2=n2, boff=boff),
        grid=(bs, nt),
        in_specs=[
            pl.BlockSpec((1, 8, ta), lambda b, t: (b, 0, t)),
            pl.BlockSpec((1, n2, 8), lambda b, t: (b, 0, 0)),
        ],
        out_specs=[
            pl.BlockSpec((1, 8, ta), lambda b, t, _nt=nt: (0, 0, b * _nt + t)),
            pl.BlockSpec((1, 8, ta), lambda b, t, _nt=nt: (0, 0, b * _nt + t)),
        ],
        out_shape=[
            jax.ShapeDtypeStruct((1, 8, rows), jnp.int32),
            jax.ShapeDtypeStruct((1, 8, rows), jnp.float32),
        ],
    )(xyz1p, xyz2p)


def _sc_interp_call(fea2f, idx8, w8):
    rows, c2 = idx8.shape[1], fea2f.shape[1]
    info = plsc.get_sparse_core_info()
    nc, ns = info.num_cores, info.num_subcores
    nw = nc * ns
    per_w = rows // nw
    n_chunks = per_w // SC_CHUNK
    mesh = plsc.VectorSubcoreMesh(core_axis_name="c", subcore_axis_name="s")

    @functools.partial(
        pl.kernel, mesh=mesh,
        compiler_params=pltpu.CompilerParams(needs_layout_passes=False),
        out_type=jax.ShapeDtypeStruct((rows, c2), jnp.float32),
        scratch_types=[
            pltpu.VMEM((SC_CHUNK,), jnp.int32),
            pltpu.VMEM((SC_CHUNK,), jnp.int32),
            pltpu.VMEM((SC_CHUNK,), jnp.int32),
            pltpu.VMEM((SC_CHUNK,), jnp.float32),
            pltpu.VMEM((SC_CHUNK,), jnp.float32),
            pltpu.VMEM((SC_CHUNK,), jnp.float32),
            pltpu.VMEM((SC_CHUNK, c2), jnp.float32),
            pltpu.VMEM((SC_CHUNK, c2), jnp.float32),
            pltpu.VMEM((SC_CHUNK, c2), jnp.float32),
            pltpu.VMEM((SC_CHUNK, c2), jnp.float32),
            pltpu.SemaphoreType.DMA,
        ],
    )
    def sc_kernel(fea2_hbm, idx_hbm, w_hbm, out_hbm,
                  i1, i2, i3, wv1, wv2, wv3, r1, r2, r3, ob, sem):
        wid = lax.axis_index("s") * nc + lax.axis_index("c")
        base0 = wid * per_w

        def chunk(ci, carry):
            base = base0 + ci * SC_CHUNK
            sl = pl.ds(base, SC_CHUNK)
            pltpu.sync_copy(idx_hbm.at[0, sl], i1)
            pltpu.sync_copy(idx_hbm.at[1, sl], i2)
            pltpu.sync_copy(idx_hbm.at[2, sl], i3)
            pltpu.sync_copy(w_hbm.at[0, sl], wv1)
            pltpu.sync_copy(w_hbm.at[1, sl], wv2)
            pltpu.sync_copy(w_hbm.at[2, sl], wv3)
            cp1 = pltpu.async_copy(fea2_hbm.at[i1], r1, sem)
            cp2 = pltpu.async_copy(fea2_hbm.at[i2], r2, sem)
            cp3 = pltpu.async_copy(fea2_hbm.at[i3], r3, sem)
            cp1.wait()
            cp2.wait()
            cp3.wait()

            @plsc.parallel_loop(0, SC_CHUNK, 1, unroll=2)
            def pair(r):
                w1 = plsc.load_gather(wv1, [jnp.full((16,), r, jnp.int32)])
                w2 = plsc.load_gather(wv2, [jnp.full((16,), r, jnp.int32)])
                w3 = plsc.load_gather(wv3, [jnp.full((16,), r, jnp.int32)])
                for c in range(c2 // 16):
                    fsl = pl.ds(c * 16, 16)
                    acc = w1 * r1[r, fsl] + w2 * r2[r, fsl] + w3 * r3[r, fsl]
                    ob[r, fsl] = acc
            pltpu.sync_copy(ob, out_hbm.at[pl.ds(base, SC_CHUNK)])
            return carry

        lax.fori_loop(0, n_chunks, chunk, 0)

    return sc_kernel(fea2f, idx8, w8)


def _mlp1_body(f1_ref, it0_ref, it1_ref, it2_ref, it3_ref,
               w1a_ref, w1b_ref, b1_ref, y_ref, st_ref, it_scr):
    g = pl.program_id(0)
    t = pl.program_id(1)
    its = [it0_ref, it1_ref, it2_ref, it3_ref]
    for gi in range(4):
        @pl.when(g == gi)
        def _(gi=gi):
            it_scr[...] = its[gi][...]
    it = it_scr[...]
    y = (lax.dot_general(f1_ref[...], w1a_ref[...], (((1,), (0,)), ((), ())),
                         preferred_element_type=jnp.float32)
         + lax.dot_general(it, w1b_ref[...], (((1,), (0,)), ((), ())),
                           preferred_element_type=jnp.float32)
         + b1_ref[...])
    y_ref[...] = y.astype(jnp.bfloat16)
    s = jnp.sum(y, axis=0, keepdims=True)
    q = jnp.sum(y * y, axis=0, keepdims=True)
    blk = jnp.concatenate(
        [s, q, jnp.zeros((6, y.shape[1]), jnp.float32)], axis=0)

    @pl.when((g == 0) & (t == 0))
    def _():
        st_ref[...] = blk

    @pl.when((g > 0) | (t > 0))
    def _():
        st_ref[...] += blk


def _mlp1_call(f1, interps, w1at, w1bt, b1r):
    rows, c1 = f1.shape
    c2 = interps[0].shape[1]
    co = w1at.shape[1]
    ng = len(interps)
    tm = min(TILE_M, interps[0].shape[0])
    ntg = interps[0].shape[0] // tm

    def piece_map(gi):
        def m(g, t):
            return (jnp.where(g == gi, t, jnp.where(g < gi, 0, ntg - 1)), 0)
        return m

    return pl.pallas_call(
        _mlp1_body,
        grid=(ng, ntg),
        in_specs=[
            pl.BlockSpec((tm, c1), lambda g, t: (g * ntg + t, 0)),
            pl.BlockSpec((tm, c2), piece_map(0)),
            pl.BlockSpec((tm, c2), piece_map(1)),
            pl.BlockSpec((tm, c2), piece_map(2)),
            pl.BlockSpec((tm, c2), piece_map(3)),
            pl.BlockSpec((c1, co), lambda g, t: (0, 0)),
            pl.BlockSpec((c2, co), lambda g, t: (0, 0)),
            pl.BlockSpec((1, co), lambda g, t: (0, 0)),
        ],
        out_specs=[
            pl.BlockSpec((tm, co), lambda g, t: (g * ntg + t, 0)),
            pl.BlockSpec((8, co), lambda g, t: (0, 0)),
        ],
        out_shape=[
            jax.ShapeDtypeStruct((rows, co), jnp.bfloat16),
            jax.ShapeDtypeStruct((8, co), jnp.float32),
        ],
        scratch_shapes=[pltpu.VMEM((tm, c2), jnp.float32)],
    )(f1, *interps, w1at, w1bt, b1r)


def _mlp2_body(y1_ref, st1_ref, g1_ref, be1_ref, w2t_ref, b2_ref,
               y2_ref, st_ref, *, n_rows):
    t = pl.program_id(0)
    st = st1_ref[...]
    m = st[0:1] / n_rows
    v = st[1:2] / n_rows - m * m
    a = g1_ref[...] / jnp.sqrt(v + 1e-5)
    y1 = y1_ref[...].astype(jnp.float32)
    x2 = jnp.maximum((y1 - m) * a + be1_ref[...], 0.0)
    y2 = lax.dot_general(x2, w2t_ref[...], (((1,), (0,)), ((), ())),
                         preferred_element_type=jnp.float32) + b2_ref[...]
    y2_ref[...] = y2.astype(jnp.bfloat16)
    s = jnp.sum(y2, axis=0, keepdims=True)
    q = jnp.sum(y2 * y2, axis=0, keepdims=True)
    blk = jnp.concatenate(
        [s, q, jnp.zeros((6, y2.shape[1]), jnp.float32)], axis=0)

    @pl.when(t == 0)
    def _():
        st_ref[...] = blk

    @pl.when(t > 0)
    def _():
        st_ref[...] += blk


def _mlp2_call(y1, st1, g1r, be1r, w2t, b2r):
    rows, ci = y1.shape
    co = w2t.shape[1]
    tm = min(TILE_M, rows)
    nt = rows // tm
    return pl.pallas_call(
        functools.partial(_mlp2_body, n_rows=float(rows)),
        grid=(nt,),
        in_specs=[
            pl.BlockSpec((tm, ci), lambda t: (t, 0)),
            pl.BlockSpec((8, ci), lambda t: (0, 0)),
            pl.BlockSpec((1, ci), lambda t: (0, 0)),
            pl.BlockSpec((1, ci), lambda t: (0, 0)),
            pl.BlockSpec((ci, co), lambda t: (0, 0)),
            pl.BlockSpec((1, co), lambda t: (0, 0)),
        ],
        out_specs=[
            pl.BlockSpec((tm, co), lambda t: (t, 0)),
            pl.BlockSpec((8, co), lambda t: (0, 0)),
        ],
        out_shape=[
            jax.ShapeDtypeStruct((rows, co), jnp.bfloat16),
            jax.ShapeDtypeStruct((8, co), jnp.float32),
        ],
    )(y1, st1, g1r, be1r, w2t, b2r)


def _norm_body(y2_ref, st2_ref, g2_ref, be2_ref, o_ref, *, n_rows):
    st = st2_ref[...]
    m = st[0:1] / n_rows
    v = st[1:2] / n_rows - m * m
    a = g2_ref[...] / jnp.sqrt(v + 1e-5)
    y2 = y2_ref[...].astype(jnp.float32)
    o_ref[...] = jnp.maximum((y2 - m) * a + be2_ref[...], 0.0)


def _norm_call(y2, st2, g2r, be2r):
    rows, co = y2.shape
    tm = min(TILE_M, rows)
    nt = rows // tm
    return pl.pallas_call(
        functools.partial(_norm_body, n_rows=float(rows)),
        grid=(nt,),
        in_specs=[
            pl.BlockSpec((tm, co), lambda t: (t, 0)),
            pl.BlockSpec((8, co), lambda t: (0, 0)),
            pl.BlockSpec((1, co), lambda t: (0, 0)),
            pl.BlockSpec((1, co), lambda t: (0, 0)),
        ],
        out_specs=pl.BlockSpec((tm, co), lambda t: (t, 0)),
        out_shape=jax.ShapeDtypeStruct((rows, co), jnp.float32),
    )(y2, st2, g2r, be2r)


def kernel(xyz1, xyz2, fea1, fea2, W1, b1, g1, be1, W2, b2, g2, be2):
    bs, n1, _ = xyz1.shape
    n2 = xyz2.shape[1]
    c1 = fea1.shape[2]
    c2 = fea2.shape[2]
    rows = bs * n1

    xyz1p = jnp.concatenate(
        [jnp.transpose(xyz1, (0, 2, 1)),
         jnp.zeros((bs, 5, n1), jnp.float32)], axis=1)
    xyz2p = jnp.concatenate(
        [xyz2, jnp.zeros((bs, n2, 5), jnp.float32)], axis=2)
    fea2f = fea2.reshape(bs * n2, c2)

    ng = 4
    bg = bs // ng
    interps = []
    for g in range(ng):
        sl = slice(g * bg, (g + 1) * bg)
        idx8, w8 = _knn_call(xyz1p[sl], xyz2p[sl], g * bg)
        interps.append(_sc_interp_call(fea2f, idx8[0], w8[0]))

    f1 = fea1.reshape(rows, c1)
    w1at = W1[:, :c1].T
    w1bt = W1[:, c1:].T
    y1, st1 = _mlp1_call(f1, interps, w1at, w1bt, b1.reshape(1, -1))
    y2, st2 = _mlp2_call(y1, st1, g1.reshape(1, -1), be1.reshape(1, -1),
                         W2.T, b2.reshape(1, -1))
    out = _norm_call(y2, st2, g2.reshape(1, -1), be2.reshape(1, -1))
    return out.reshape(bs, n1, W2.shape[0])


# speedup vs baseline: 40.3988x; 1.0985x over previous
import functools

import jax
import jax.numpy as jnp
from jax import lax
from jax.experimental import pallas as pl
from jax.experimental.pallas import tpu as pltpu
from jax.experimental.pallas import tpu_sc as plsc

TILE_A = 1024
TILE_M = 2048
SC_CHUNK = 64


def _knn_body(x1_ref, x2_ref, idx_ref, w_ref, *, n2, boff):
    b = pl.program_id(0) + boff
    x1 = x1_ref[0]
    x2 = x2_ref[0]
    dot = lax.dot_general(x2, x1, (((1,), (0,)), ((), ())),
                          preferred_element_type=jnp.float32)
    n1sq = jnp.sum(x1 * x1, axis=0, keepdims=True)
    n2sq = jnp.sum(x2 * x2, axis=1, keepdims=True)
    d = n1sq + n2sq - 2.0 * dot
    t_cols = d.shape[1]
    cand = lax.broadcasted_iota(jnp.int32, (n2, t_cols), 0).astype(jnp.float32)
    big = jnp.float32(jnp.inf)
    bigc = jnp.float32(n2)

    def minarg(dd):
        m = jnp.min(dd, axis=0, keepdims=True)
        a = jnp.min(jnp.where(dd == m, cand, bigc), axis=0, keepdims=True)
        return m, a

    m1, a1f = minarg(d)
    d2 = jnp.where(cand == a1f, big, d)
    m2, a2f = minarg(d2)
    d3 = jnp.where(cand == a2f, big, d2)
    m3, a3f = minarg(d3)
    a1 = a1f.astype(jnp.int32)
    a2 = a2f.astype(jnp.int32)
    a3 = a3f.astype(jnp.int32)

    r1 = 1.0 / (m1 + 1e-8)
    r2 = 1.0 / (m2 + 1e-8)
    r3 = 1.0 / (m3 + 1e-8)
    s = r1 + r2 + r3
    w1, w2, w3 = r1 / s, r2 / s, r3 / s

    g = b * n2
    zi = jnp.zeros((5, t_cols), jnp.int32)
    zf = jnp.zeros((5, t_cols), jnp.float32)
    idx_ref[0] = jnp.concatenate([a1 + g, a2 + g, a3 + g, zi], axis=0)
    w_ref[0] = jnp.concatenate([w1, w2, w3, zf], axis=0)


def _knn_call(xyz1p, xyz2p, boff):
    bs, n1 = xyz1p.shape[0], xyz1p.shape[2]
    n2 = xyz2p.shape[1]
    ta = min(TILE_A, n1)
    nt = n1 // ta
    rows = bs * n1
    return pl.pallas_call(
        functools.partial(_knn_body, n2=n2, boff=boff),
        grid=(bs, nt),
        in_specs=[
            pl.BlockSpec((1, 8, ta), lambda b, t: (b, 0, t)),
            pl.BlockSpec((1, n2, 8), lambda b, t: (b, 0, 0)),
        ],
        out_specs=[
            pl.BlockSpec((1, 8, ta), lambda b, t, _nt=nt: (0, 0, b * _nt + t)),
            pl.BlockSpec((1, 8, ta), lambda b, t, _nt=nt: (0, 0, b * _nt + t)),
        ],
        out_shape=[
            jax.ShapeDtypeStruct((1, 8, rows), jnp.int32),
            jax.ShapeDtypeStruct((1, 8, rows), jnp.float32),
        ],
    )(xyz1p, xyz2p)


def _sc_interp_call(fea2f, idx8, w8):
    rows, c2 = idx8.shape[1], fea2f.shape[1]
    info = plsc.get_sparse_core_info()
    nc, ns = info.num_cores, info.num_subcores
    nw = nc * ns
    per_w = rows // nw
    n_chunks = per_w // SC_CHUNK
    mesh = plsc.VectorSubcoreMesh(core_axis_name="c", subcore_axis_name="s")

    @functools.partial(
        pl.kernel, mesh=mesh,
        compiler_params=pltpu.CompilerParams(needs_layout_passes=False),
        out_type=jax.ShapeDtypeStruct((rows, c2), jnp.float32),
        scratch_types=[
            pltpu.VMEM((SC_CHUNK,), jnp.int32),
            pltpu.VMEM((SC_CHUNK,), jnp.int32),
            pltpu.VMEM((SC_CHUNK,), jnp.int32),
            pltpu.VMEM((SC_CHUNK,), jnp.float32),
            pltpu.VMEM((SC_CHUNK,), jnp.float32),
            pltpu.VMEM((SC_CHUNK,), jnp.float32),
            pltpu.VMEM((SC_CHUNK, c2), jnp.float32),
            pltpu.VMEM((SC_CHUNK, c2), jnp.float32),
            pltpu.VMEM((SC_CHUNK, c2), jnp.float32),
            pltpu.VMEM((SC_CHUNK, c2), jnp.float32),
            pltpu.SemaphoreType.DMA,
        ],
    )
    def sc_kernel(fea2_hbm, idx_hbm, w_hbm, out_hbm,
                  i1, i2, i3, wv1, wv2, wv3, r1, r2, r3, ob, sem):
        wid = lax.axis_index("s") * nc + lax.axis_index("c")
        base0 = wid * per_w

        def chunk(ci, carry):
            base = base0 + ci * SC_CHUNK
            sl = pl.ds(base, SC_CHUNK)
            pltpu.sync_copy(idx_hbm.at[0, sl], i1)
            pltpu.sync_copy(idx_hbm.at[1, sl], i2)
            pltpu.sync_copy(idx_hbm.at[2, sl], i3)
            pltpu.sync_copy(w_hbm.at[0, sl], wv1)
            pltpu.sync_copy(w_hbm.at[1, sl], wv2)
            pltpu.sync_copy(w_hbm.at[2, sl], wv3)
            cp1 = pltpu.async_copy(fea2_hbm.at[i1], r1, sem)
            cp2 = pltpu.async_copy(fea2_hbm.at[i2], r2, sem)
            cp3 = pltpu.async_copy(fea2_hbm.at[i3], r3, sem)
            cp1.wait()
            cp2.wait()
            cp3.wait()

            @plsc.parallel_loop(0, SC_CHUNK, 1, unroll=2)
            def pair(r):
                w1 = plsc.load_gather(wv1, [jnp.full((16,), r, jnp.int32)])
                w2 = plsc.load_gather(wv2, [jnp.full((16,), r, jnp.int32)])
                w3 = plsc.load_gather(wv3, [jnp.full((16,), r, jnp.int32)])
                for c in range(c2 // 16):
                    fsl = pl.ds(c * 16, 16)
                    acc = w1 * r1[r, fsl] + w2 * r2[r, fsl] + w3 * r3[r, fsl]
                    ob[r, fsl] = acc
            pltpu.sync_copy(ob, out_hbm.at[pl.ds(base, SC_CHUNK)])
            return carry

        lax.fori_loop(0, n_chunks, chunk, 0)

    return sc_kernel(fea2f, idx8, w8)


def _mlp1_body(f1_ref, it0_ref, it1_ref, it2_ref, it3_ref,
               w1a_ref, w1b_ref, b1_ref, y_ref, st_ref, it_scr):
    g = pl.program_id(0)
    t = pl.program_id(1)
    its = [it0_ref, it1_ref, it2_ref, it3_ref]
    for gi in range(4):
        @pl.when(g == gi)
        def _(gi=gi):
            it_scr[...] = its[gi][...]
    it = it_scr[...]
    y = (lax.dot_general(f1_ref[...], w1a_ref[...], (((1,), (0,)), ((), ())),
                         preferred_element_type=jnp.float32)
         + lax.dot_general(it, w1b_ref[...], (((1,), (0,)), ((), ())),
                           preferred_element_type=jnp.float32)
         + b1_ref[...])
    y_ref[...] = y.astype(jnp.bfloat16)
    s = jnp.sum(y, axis=0, keepdims=True)
    q = jnp.sum(y * y, axis=0, keepdims=True)
    blk = jnp.concatenate(
        [s, q, jnp.zeros((6, y.shape[1]), jnp.float32)], axis=0)

    @pl.when((g == 0) & (t == 0))
    def _():
        st_ref[...] = blk

    @pl.when((g > 0) | (t > 0))
    def _():
        st_ref[...] += blk


def _mlp1_call(f1, interps, w1at, w1bt, b1r):
    rows, c1 = f1.shape
    c2 = interps[0].shape[1]
    co = w1at.shape[1]
    ng = len(interps)
    tm = min(TILE_M, interps[0].shape[0])
    ntg = interps[0].shape[0] // tm

    def piece_map(gi):
        def m(g, t):
            return (jnp.where(g == gi, t, jnp.where(g < gi, 0, ntg - 1)), 0)
        return m

    return pl.pallas_call(
        _mlp1_body,
        grid=(ng, ntg),
        in_specs=[
            pl.BlockSpec((tm, c1), lambda g, t: (g * ntg + t, 0)),
            pl.BlockSpec((tm, c2), piece_map(0)),
            pl.BlockSpec((tm, c2), piece_map(1)),
            pl.BlockSpec((tm, c2), piece_map(2)),
            pl.BlockSpec((tm, c2), piece_map(3)),
            pl.BlockSpec((c1, co), lambda g, t: (0, 0)),
            pl.BlockSpec((c2, co), lambda g, t: (0, 0)),
            pl.BlockSpec((1, co), lambda g, t: (0, 0)),
        ],
        out_specs=[
            pl.BlockSpec((tm, co), lambda g, t: (g * ntg + t, 0)),
            pl.BlockSpec((8, co), lambda g, t: (0, 0)),
        ],
        out_shape=[
            jax.ShapeDtypeStruct((rows, co), jnp.bfloat16),
            jax.ShapeDtypeStruct((8, co), jnp.float32),
        ],
        scratch_shapes=[pltpu.VMEM((tm, c2), jnp.float32)],
    )(f1, *interps, w1at, w1bt, b1r)


def _mlp2_body(y1_ref, st1_ref, g1_ref, be1_ref, w2t_ref, b2_ref,
               y2_ref, st_ref, *, n_rows):
    t = pl.program_id(0)
    st = st1_ref[...]
    m = st[0:1] / n_rows
    v = st[1:2] / n_rows - m * m
    a = g1_ref[...] / jnp.sqrt(v + 1e-5)
    y1 = y1_ref[...].astype(jnp.float32)
    x2 = jnp.maximum((y1 - m) * a + be1_ref[...], 0.0)
    y2 = lax.dot_general(x2, w2t_ref[...], (((1,), (0,)), ((), ())),
                         preferred_element_type=jnp.float32) + b2_ref[...]
    y2_ref[...] = y2.astype(jnp.bfloat16)
    s = jnp.sum(y2, axis=0, keepdims=True)
    q = jnp.sum(y2 * y2, axis=0, keepdims=True)
    blk = jnp.concatenate(
        [s, q, jnp.zeros((6, y2.shape[1]), jnp.float32)], axis=0)

    @pl.when(t == 0)
    def _():
        st_ref[...] = blk

    @pl.when(t > 0)
    def _():
        st_ref[...] += blk


def _mlp2_call(y1, st1, g1r, be1r, w2t, b2r):
    rows, ci = y1.shape
    co = w2t.shape[1]
    tm = min(TILE_M, rows)
    nt = rows // tm
    return pl.pallas_call(
        functools.partial(_mlp2_body, n_rows=float(rows)),
        grid=(nt,),
        in_specs=[
            pl.BlockSpec((tm, ci), lambda t: (t, 0)),
            pl.BlockSpec((8, ci), lambda t: (0, 0)),
            pl.BlockSpec((1, ci), lambda t: (0, 0)),
            pl.BlockSpec((1, ci), lambda t: (0, 0)),
            pl.BlockSpec((ci, co), lambda t: (0, 0)),
            pl.BlockSpec((1, co), lambda t: (0, 0)),
        ],
        out_specs=[
            pl.BlockSpec((tm, co), lambda t: (t, 0)),
            pl.BlockSpec((8, co), lambda t: (0, 0)),
        ],
        out_shape=[
            jax.ShapeDtypeStruct((rows, co), jnp.bfloat16),
            jax.ShapeDtypeStruct((8, co), jnp.float32),
        ],
    )(y1, st1, g1r, be1r, w2t, b2r)


def _norm_body(y2_ref, st2_ref, g2_ref, be2_ref, o_ref, *, n_rows):
    st = st2_ref[...]
    m = st[0:1] / n_rows
    v = st[1:2] / n_rows - m * m
    a = g2_ref[...] / jnp.sqrt(v + 1e-5)
    y2 = y2_ref[...].astype(jnp.float32)
    o_ref[...] = jnp.maximum((y2 - m) * a + be2_ref[...], 0.0)


def _norm_call(y2, st2, g2r, be2r):
    rows, co = y2.shape
    tm = min(TILE_M, rows)
    nt = rows // tm
    return pl.pallas_call(
        functools.partial(_norm_body, n_rows=float(rows)),
        grid=(nt,),
        in_specs=[
            pl.BlockSpec((tm, co), lambda t: (t, 0)),
            pl.BlockSpec((8, co), lambda t: (0, 0)),
            pl.BlockSpec((1, co), lambda t: (0, 0)),
            pl.BlockSpec((1, co), lambda t: (0, 0)),
        ],
        out_specs=pl.BlockSpec((tm, co), lambda t: (t, 0)),
        out_shape=jax.ShapeDtypeStruct((rows, co), jnp.float32),
    )(y2, st2, g2r, be2r)


def kernel(xyz1, xyz2, fea1, fea2, W1, b1, g1, be1, W2, b2, g2, be2):
    bs, n1, _ = xyz1.shape
    n2 = xyz2.shape[1]
    c1 = fea1.shape[2]
    c2 = fea2.shape[2]
    rows = bs * n1

    xyz1p = jnp.concatenate(
        [jnp.transpose(xyz1, (0, 2, 1)),
         jnp.zeros((bs, 5, n1), jnp.float32)], axis=1)
    xyz2p = jnp.concatenate(
        [xyz2, jnp.zeros((bs, n2, 5), jnp.float32)], axis=2)
    fea2f = fea2.reshape(bs * n2, c2)

    ng = 4
    bg = bs // ng
    interps = []
    for g in range(ng):
        sl = slice(g * bg, (g + 1) * bg)
        idx8, w8 = _knn_call(xyz1p[sl], xyz2p[sl], g * bg)
        interps.append(_sc_interp_call(fea2f, idx8[0], w8[0]))

    f1 = fea1.reshape(rows, c1)
    w1at = W1[:, :c1].T
    w1bt = W1[:, c1:].T
    y1, st1 = _mlp1_call(f1, interps, w1at, w1bt, b1.reshape(1, -1))
    y2, st2 = _mlp2_call(y1, st1, g1.reshape(1, -1), be1.reshape(1, -1),
                         W2.T, b2.reshape(1, -1))
    out = _norm_call(y2, st2, g2.reshape(1, -1), be2.reshape(1, -1))
    return out.reshape(bs, n1, W2.shape[0])
